# Initial kernel scaffold; baseline (speedup 1.0000x reference)
import jax
import jax.numpy as jnp
from jax.experimental import pallas as pl


def kernel(atoms, degrees, atom_table, degree_table):
    raise NotImplementedError("write your pallas kernel here")


# speedup vs baseline: 2.5969x; 2.5969x over previous
import functools

import jax
import jax.numpy as jnp
from jax import lax
from jax.experimental import pallas as pl
from jax.experimental.pallas import tpu as pltpu
from jax.experimental.pallas import tpu_sc as plsc

ATOM_TYPES = 10
MAX_DEGREE = 64
DIM = 64

_NC = 2
_NS = 16
_NW = _NC * _NS

_T = 512


def _sc_body(atoms_hbm, degrees_hbm, atom_hbm, deg_hbm, out_hbm,
             atom_v, deg_v, comb_v, ai_v, di_v, out_v, *, per_w, chunks):
  wid = lax.axis_index("s") * _NC + lax.axis_index("c")

  pltpu.sync_copy(atom_hbm, atom_v)
  pltpu.sync_copy(deg_hbm, deg_v)

  for a in range(ATOM_TYPES):
    a_rows = [atom_v[pl.ds(a * DIM + q * 16, 16)] for q in range(4)]

    def build_d(d, _, a=a, a_rows=a_rows):
      base = (a * MAX_DEGREE + d) * DIM
      for q in range(4):
        comb_v[pl.ds(base + q * 16, 16)] = (
            a_rows[q] + deg_v[pl.ds(d * DIM + q * 16, 16)])
      return 0

    lax.fori_loop(0, MAX_DEGREE, build_d, 0)

  iota = lax.iota(jnp.int32, 16)
  obase = iota * DIM
  w0 = wid * per_w

  def chunk_body(g, _):
    tok0 = w0 + g * _T
    pltpu.sync_copy(atoms_hbm.at[pl.ds(tok0, _T)], ai_v)
    pltpu.sync_copy(degrees_hbm.at[pl.ds(tok0, _T)], di_v)

    def grp(j, _):
      av = ai_v[pl.ds(j * 16, 16)]
      dv = di_v[pl.ds(j * 16, 16)]
      key = av * (MAX_DEGREE * DIM) + dv * DIM
      oix = obase + j * (16 * DIM)
      for c in range(DIM):
        vals = plsc.load_gather(comb_v, [key + c])
        plsc.store_scatter(out_v, [oix + c], vals)
      return 0

    lax.fori_loop(0, _T // 16, grp, 0)
    pltpu.sync_copy(out_v, out_hbm.at[pl.ds(tok0 * DIM, _T * DIM)])
    return 0

  lax.fori_loop(0, chunks, chunk_body, 0)


def kernel(atoms, degrees, atom_table, degree_table):
  B, L = atoms.shape
  n = B * L
  per_w = n // _NW
  chunks = per_w // _T
  assert per_w * _NW == n and chunks * _T == per_w

  mesh = plsc.VectorSubcoreMesh(core_axis_name="c", subcore_axis_name="s")
  body = functools.partial(_sc_body, per_w=per_w, chunks=chunks)
  out_flat = pl.kernel(
      body,
      out_type=jax.ShapeDtypeStruct((n * DIM,), jnp.float32),
      mesh=mesh,
      compiler_params=pltpu.CompilerParams(needs_layout_passes=False),
      scratch_types=[
          pltpu.VMEM((ATOM_TYPES * DIM,), jnp.float32),
          pltpu.VMEM((MAX_DEGREE * DIM,), jnp.float32),
          pltpu.VMEM((ATOM_TYPES * MAX_DEGREE * DIM,), jnp.float32),
          pltpu.VMEM((_T,), jnp.int32),
          pltpu.VMEM((_T,), jnp.int32),
          pltpu.VMEM((_T * DIM,), jnp.float32),
      ],
  )(
      atoms.reshape(-1).astype(jnp.int32),
      degrees.reshape(-1).astype(jnp.int32),
      atom_table.reshape(-1),
      degree_table.reshape(-1),
  )
  return out_flat.reshape(B, L, DIM)


# speedup vs baseline: 3.4220x; 1.3178x over previous
import functools

import jax
import jax.numpy as jnp
from jax import lax
from jax.experimental import pallas as pl
from jax.experimental.pallas import tpu as pltpu
from jax.experimental.pallas import tpu_sc as plsc

ATOM_TYPES = 10
MAX_DEGREE = 64
DIM = 64

_NC = 2
_NS = 16
_NW = _NC * _NS

_T = 512


def _sc_body(atoms_hbm, degrees_hbm, atom_hbm, deg_hbm, out_hbm,
             atom_v, deg_v, comb_v, ai_v, di_v, out_v, *, per_w, chunks):
  wid = lax.axis_index("s") * _NC + lax.axis_index("c")

  pltpu.sync_copy(atom_hbm, atom_v)
  pltpu.sync_copy(deg_hbm, deg_v)

  for a in range(ATOM_TYPES):
    a_rows = [atom_v[pl.ds(a * DIM + q * 16, 16)] for q in range(4)]

    def build_d(d, _, a=a, a_rows=a_rows):
      base = (a * MAX_DEGREE + d) * DIM
      for q in range(4):
        comb_v[pl.ds(base + q * 16, 16)] = (
            a_rows[q] + deg_v[pl.ds(d * DIM + q * 16, 16)])
      return 0

    lax.fori_loop(0, MAX_DEGREE, build_d, 0)

  iota = lax.iota(jnp.int32, 16)
  obase = iota * DIM
  w0 = wid * per_w

  def chunk_body(g, _):
    tok0 = w0 + g * _T
    pltpu.sync_copy(atoms_hbm.at[pl.ds(tok0, _T)], ai_v)
    pltpu.sync_copy(degrees_hbm.at[pl.ds(tok0, _T)], di_v)

    @plsc.parallel_loop(0, _T // 16, unroll=2)
    def grp(j):
      av = ai_v[pl.ds(j * 16, 16)]
      dv = di_v[pl.ds(j * 16, 16)]
      key = av * (MAX_DEGREE * DIM) + dv * DIM
      oix = obase + j * (16 * DIM)
      ncomb = ATOM_TYPES * MAX_DEGREE * DIM
      nout = _T * DIM
      key_r = [key + r for r in range(8)]
      oix_r = [oix + r for r in range(8)]
      for cc in range(0, DIM, 16):
        cols = range(cc, cc + 16)
        vals = [plsc.load_gather(
                    comb_v.at[pl.ds((c // 8) * 8, ncomb - (c // 8) * 8)],
                    [key_r[c % 8]])
                for c in cols]
        for c, v in zip(cols, vals):
          plsc.store_scatter(
              out_v.at[pl.ds((c // 8) * 8, nout - (c // 8) * 8)],
              [oix_r[c % 8]], v)
    pltpu.sync_copy(out_v, out_hbm.at[pl.ds(tok0 * DIM, _T * DIM)])
    return 0

  lax.fori_loop(0, chunks, chunk_body, 0)


def kernel(atoms, degrees, atom_table, degree_table):
  B, L = atoms.shape
  n = B * L
  per_w = n // _NW
  chunks = per_w // _T
  assert per_w * _NW == n and chunks * _T == per_w

  mesh = plsc.VectorSubcoreMesh(core_axis_name="c", subcore_axis_name="s")
  body = functools.partial(_sc_body, per_w=per_w, chunks=chunks)
  out_flat = pl.kernel(
      body,
      out_type=jax.ShapeDtypeStruct((n * DIM,), jnp.float32),
      mesh=mesh,
      compiler_params=pltpu.CompilerParams(needs_layout_passes=False),
      scratch_types=[
          pltpu.VMEM((ATOM_TYPES * DIM,), jnp.float32),
          pltpu.VMEM((MAX_DEGREE * DIM,), jnp.float32),
          pltpu.VMEM((ATOM_TYPES * MAX_DEGREE * DIM,), jnp.float32),
          pltpu.VMEM((_T,), jnp.int32),
          pltpu.VMEM((_T,), jnp.int32),
          pltpu.VMEM((_T * DIM,), jnp.float32),
      ],
  )(
      atoms.reshape(-1).astype(jnp.int32),
      degrees.reshape(-1).astype(jnp.int32),
      atom_table.reshape(-1),
      degree_table.reshape(-1),
  )
  return out_flat.reshape(B, L, DIM)


# speedup vs baseline: 9.2148x; 2.6928x over previous
import functools

import jax
import jax.numpy as jnp
from jax import lax
from jax.experimental import pallas as pl
from jax.experimental.pallas import tpu as pltpu
from jax.experimental.pallas import tpu_sc as plsc

ATOM_TYPES = 10
MAX_DEGREE = 64
DIM = 64

_NC = 2
_NS = 16
_NW = _NC * _NS

_T = 512


def _sc_body(atoms_hbm, degrees_hbm, atom_hbm, deg_hbm, out_hbm,
             atom_v, deg_v, comb_v, ai_v, di_v, key_v, out_v,
             *, per_w, chunks):
  wid = lax.axis_index("s") * _NC + lax.axis_index("c")

  pltpu.sync_copy(atom_hbm, atom_v)
  pltpu.sync_copy(deg_hbm, deg_v)

  for a in range(ATOM_TYPES):
    a_rows = [atom_v[pl.ds(a * DIM + q * 16, 16)] for q in range(4)]

    def build_d(d, _, a=a, a_rows=a_rows):
      base = (a * MAX_DEGREE + d) * DIM
      for q in range(4):
        comb_v[pl.ds(base + q * 16, 16)] = (
            a_rows[q] + deg_v[pl.ds(d * DIM + q * 16, 16)])
      return 0

    lax.fori_loop(0, MAX_DEGREE, build_d, 0)

  w0 = wid * per_w

  def chunk_body(g, _):
    tok0 = w0 + g * _T
    pltpu.sync_copy(atoms_hbm.at[pl.ds(tok0, _T)], ai_v)
    pltpu.sync_copy(degrees_hbm.at[pl.ds(tok0, _T)], di_v)

    @plsc.parallel_loop(0, _T // 16, unroll=2)
    def keys(j):
      av = ai_v[pl.ds(j * 16, 16)]
      dv = di_v[pl.ds(j * 16, 16)]
      key_v[pl.ds(j * 16, 16)] = av * (MAX_DEGREE * DIM) + dv * DIM

    @plsc.parallel_loop(0, _T // 16, unroll=1)
    def rows(j):
      kvec = key_v[pl.ds(j * 16, 16)]
      for i in range(16):
        k = kvec[i]
        t16 = j * (16 * DIM) + i * DIM
        for q in range(DIM // 16):
          out_v[pl.ds(t16 + q * 16, 16)] = comb_v[pl.ds(k + q * 16, 16)]

    pltpu.sync_copy(out_v, out_hbm.at[pl.ds(tok0 * DIM, _T * DIM)])
    return 0

  lax.fori_loop(0, chunks, chunk_body, 0)


def kernel(atoms, degrees, atom_table, degree_table):
  B, L = atoms.shape
  n = B * L
  per_w = n // _NW
  chunks = per_w // _T
  assert per_w * _NW == n and chunks * _T == per_w

  mesh = plsc.VectorSubcoreMesh(core_axis_name="c", subcore_axis_name="s")
  body = functools.partial(_sc_body, per_w=per_w, chunks=chunks)
  out_flat = pl.kernel(
      body,
      out_type=jax.ShapeDtypeStruct((n * DIM,), jnp.float32),
      mesh=mesh,
      compiler_params=pltpu.CompilerParams(needs_layout_passes=False),
      scratch_types=[
          pltpu.VMEM((ATOM_TYPES * DIM,), jnp.float32),
          pltpu.VMEM((MAX_DEGREE * DIM,), jnp.float32),
          pltpu.VMEM((ATOM_TYPES * MAX_DEGREE * DIM,), jnp.float32),
          pltpu.VMEM((_T,), jnp.int32),
          pltpu.VMEM((_T,), jnp.int32),
          pltpu.VMEM((_T,), jnp.int32),
          pltpu.VMEM((_T * DIM,), jnp.float32),
      ],
  )(
      atoms.reshape(-1).astype(jnp.int32),
      degrees.reshape(-1).astype(jnp.int32),
      atom_table.reshape(-1),
      degree_table.reshape(-1),
  )
  return out_flat.reshape(B, L, DIM)


# speedup vs baseline: 10.8976x; 1.1826x over previous
import functools

import jax
import jax.numpy as jnp
from jax import lax
from jax.experimental import pallas as pl
from jax.experimental.pallas import tpu as pltpu
from jax.experimental.pallas import tpu_sc as plsc

ATOM_TYPES = 10
MAX_DEGREE = 64
DIM = 64

_NC = 2
_NS = 16
_NW = _NC * _NS

_T = 512


def _sc_body(atoms_hbm, degrees_hbm, atom_hbm, deg_hbm, out_hbm,
             atom_v, deg_v, comb_v, ai_v, di_v, key_v, out_v,
             sem_ai, sem_di, sem_out, *, per_w, chunks):
  wid = lax.axis_index("s") * _NC + lax.axis_index("c")
  w0 = wid * per_w

  pltpu.sync_copy(atom_hbm, atom_v)
  pltpu.sync_copy(deg_hbm, deg_v)

  for a in range(ATOM_TYPES):
    a_rows = [atom_v[pl.ds(a * DIM + q * 16, 16)] for q in range(4)]

    def build_d(d, _, a=a, a_rows=a_rows):
      base = (a * MAX_DEGREE + d) * DIM
      for q in range(4):
        comb_v[pl.ds(base + q * 16, 16)] = (
            a_rows[q] + deg_v[pl.ds(d * DIM + q * 16, 16)])
      return 0

    lax.fori_loop(0, MAX_DEGREE, build_d, 0)

  def start_in(g, b):
    t0 = w0 + g * _T
    pltpu.async_copy(atoms_hbm.at[pl.ds(t0, _T)], ai_v[b], sem_ai[b])
    pltpu.async_copy(degrees_hbm.at[pl.ds(t0, _T)], di_v[b], sem_di[b])

  def wait_in(b):
    pltpu.make_async_copy(atoms_hbm.at[pl.ds(0, _T)], ai_v[b],
                          sem_ai[b]).wait()
    pltpu.make_async_copy(degrees_hbm.at[pl.ds(0, _T)], di_v[b],
                          sem_di[b]).wait()

  def start_out(g, b):
    t0 = w0 + g * _T
    pltpu.async_copy(out_v[b], out_hbm.at[pl.ds(t0 * DIM, _T * DIM)],
                     sem_out[b])

  def wait_out(b):
    pltpu.make_async_copy(out_v[b], out_hbm.at[pl.ds(0, _T * DIM)],
                          sem_out[b]).wait()

  start_in(0, 0)
  start_in(1, 1)

  def outer(i, _):
    g0 = i * 2
    for b in range(2):
      g = g0 + b
      wait_in(b)

      @plsc.parallel_loop(0, _T // 16, unroll=2)
      def keys(j):
        av = ai_v[b][pl.ds(j * 16, 16)]
        dv = di_v[b][pl.ds(j * 16, 16)]
        key_v[pl.ds(j * 16, 16)] = av * (MAX_DEGREE * DIM) + dv * DIM

      @pl.when(g + 2 < chunks)
      def _():
        start_in(g + 2, b)

      @pl.when(g >= 2)
      def _():
        wait_out(b)

      @plsc.parallel_loop(0, _T // 16, unroll=1)
      def rows(j):
        kvec = key_v[pl.ds(j * 16, 16)]
        for i16 in range(16):
          k = kvec[i16]
          t16 = j * (16 * DIM) + i16 * DIM
          for q in range(DIM // 16):
            out_v[b][pl.ds(t16 + q * 16, 16)] = comb_v[pl.ds(k + q * 16, 16)]

      start_out(g, b)
    return 0

  lax.fori_loop(0, chunks // 2, outer, 0)
  wait_out(0)
  wait_out(1)


def kernel(atoms, degrees, atom_table, degree_table):
  B, L = atoms.shape
  n = B * L
  per_w = n // _NW
  chunks = per_w // _T
  assert per_w * _NW == n and chunks * _T == per_w and chunks % 2 == 0

  mesh = plsc.VectorSubcoreMesh(core_axis_name="c", subcore_axis_name="s")
  body = functools.partial(_sc_body, per_w=per_w, chunks=chunks)
  out_flat = pl.kernel(
      body,
      out_type=jax.ShapeDtypeStruct((n * DIM,), jnp.float32),
      mesh=mesh,
      compiler_params=pltpu.CompilerParams(needs_layout_passes=False),
      scratch_types=[
          pltpu.VMEM((ATOM_TYPES * DIM,), jnp.float32),
          pltpu.VMEM((MAX_DEGREE * DIM,), jnp.float32),
          pltpu.VMEM((ATOM_TYPES * MAX_DEGREE * DIM,), jnp.float32),
          [pltpu.VMEM((_T,), jnp.int32) for _ in range(2)],
          [pltpu.VMEM((_T,), jnp.int32) for _ in range(2)],
          pltpu.VMEM((_T,), jnp.int32),
          [pltpu.VMEM((_T * DIM,), jnp.float32) for _ in range(2)],
          [pltpu.SemaphoreType.DMA for _ in range(2)],
          [pltpu.SemaphoreType.DMA for _ in range(2)],
          [pltpu.SemaphoreType.DMA for _ in range(2)],
      ],
  )(
      atoms.reshape(-1).astype(jnp.int32),
      degrees.reshape(-1).astype(jnp.int32),
      atom_table.reshape(-1),
      degree_table.reshape(-1),
  )
  return out_flat.reshape(B, L, DIM)
